# baseline (speedup 1.0000x reference)
import functools

import jax
import jax.numpy as jnp
from jax import lax
from jax.experimental import pallas as pl
from jax.experimental.pallas import tpu as pltpu
from jax.experimental.pallas import tpu_sc as plsc

_ROWS = 128
_COLS = 32768
_K = 8
_BLOCK_ROWS = 16
_NCH = 128
_CW = 256
_CAND = _K * _CW
_NEG = float("-inf")
_BIGI = 2**30

_NC = 2
_NS = 16
_RPW = _ROWS // (_NC * _NS)


def _cm_body(x_ref, cm_ref):
    x = x_ref[...]
    cm_ref[...] = jnp.max(x.reshape(_BLOCK_ROWS, _NCH, _CW), axis=2)


def _tc_cm(x):
    return pl.pallas_call(
        _cm_body,
        grid=(_ROWS // _BLOCK_ROWS,),
        in_specs=[pl.BlockSpec((_BLOCK_ROWS, _COLS), lambda i: (i, 0))],
        out_specs=pl.BlockSpec((_BLOCK_ROWS, _NCH), lambda i: (i, 0)),
        out_shape=jax.ShapeDtypeStruct((_ROWS, _NCH), jnp.float32),
    )(x)


def _ms_body(x_ref, cm_ref, ms_ref):
    x = x_ref[...]
    m = jnp.max(cm_ref[...], axis=1, keepdims=True)
    s = jnp.sum(jnp.exp(x - m), axis=1, keepdims=True)
    ms_ref[...] = jnp.concatenate(
        [jnp.broadcast_to(m, (_BLOCK_ROWS, 16)),
         jnp.broadcast_to(s, (_BLOCK_ROWS, 16))], axis=1)


def _tc_ms(x, cm):
    return pl.pallas_call(
        _ms_body,
        grid=(_ROWS // _BLOCK_ROWS,),
        in_specs=[
            pl.BlockSpec((_BLOCK_ROWS, _COLS), lambda i: (i, 0)),
            pl.BlockSpec((_BLOCK_ROWS, _NCH), lambda i: (i, 0)),
        ],
        out_specs=pl.BlockSpec((_BLOCK_ROWS, 32), lambda i: (i, 0)),
        out_shape=jax.ShapeDtypeStruct((_ROWS, 32), jnp.float32),
    )(x, cm)


def _iota16():
    return lax.broadcasted_iota(jnp.int32, (16,), 0)


_GDN = lax.GatherDimensionNumbers(
    offset_dims=(), collapsed_slice_dims=(0,), start_index_map=(0,))


def _perm(v, s):
    idx = (_iota16() ^ s).reshape(16, 1)
    return lax.gather(v, idx, _GDN, (1,),
                      mode=lax.GatherScatterMode.PROMISE_IN_BOUNDS)


def _bfly_max(v):
    for s in (1, 2, 4, 8):
        v = jnp.maximum(v, _perm(v, s))
    return v


def _bfly_min(v):
    for s in (1, 2, 4, 8):
        v = jnp.minimum(v, _perm(v, s))
    return v


def _sc_body(cm_hbm, cids_hbm, statsbuf, ibuf):
    iota = _iota16()
    wid = lax.axis_index("s") * _NC + lax.axis_index("c")
    row0 = wid * _RPW

    pltpu.sync_copy(cm_hbm.at[pl.ds(row0 * _NCH, _RPW * _NCH)], statsbuf)

    nv = _NCH // 16

    srts = []
    for rr in range(_RPW):
        cms = [statsbuf[pl.ds(rr * _NCH + 16 * q, 16)] for q in range(nv)]
        cids_vec = jnp.full((16,), _BIGI, jnp.int32)
        for k in range(_K):
            e = cms[0]
            for q in range(1, nv):
                e = jnp.maximum(e, cms[q])
            g = _bfly_max(e)
            im = jnp.full((16,), 4 * _NCH, jnp.int32)
            for q in range(nv):
                im = jnp.minimum(im, jnp.where(cms[q] == g, iota + 16 * q,
                                               4 * _NCH))
            cid = _bfly_min(im)
            for q in range(nv):
                cms[q] = jnp.where(iota + 16 * q == cid, _NEG, cms[q])
            cids_vec = jnp.where(iota == k, cid, cids_vec)
        srt = jnp.full((16,), _BIGI, jnp.int32)
        for k in range(_K):
            c = _bfly_min(cids_vec)
            srt = jnp.where(iota == k, c, srt)
            cids_vec = jnp.where(cids_vec == c, _BIGI, cids_vec)
        srts.append(srt)

    for pair in range(_RPW // 2):
        lo, hi = srts[2 * pair], srts[2 * pair + 1]
        both = jnp.where(iota < 8, lo, _perm(hi, 8))
        ibuf[pl.ds(pair * 16, 16)] = both
    pltpu.sync_copy(ibuf, cids_hbm.at[pl.ds(row0 * _K, _RPW * _K)])


@functools.partial(
    pl.kernel,
    out_type=[jax.ShapeDtypeStruct((_ROWS * _K,), jnp.int32)],
    mesh=plsc.VectorSubcoreMesh(core_axis_name="c", subcore_axis_name="s"),
    scratch_types=[
        pltpu.VMEM((_RPW * _NCH,), jnp.float32),
        pltpu.VMEM((_RPW * _K,), jnp.int32),
    ],
)
def _sc_select(cm_hbm, cids_hbm, statsbuf, ibuf):
    _sc_body(cm_hbm, cids_hbm, statsbuf, ibuf)


_BRC = 16


def _final_body(cids_smem, x_hbm, cids_ref, ms_ref, vals_ref, idx_ref,
                cand, sems):
    pid = pl.program_id(0)
    ngrid = _ROWS // _BRC

    def chunk_descs(blk, slot):
        descs = []
        for r in range(_BRC):
            row = blk * _BRC + r
            for k in range(_K):
                cid = cids_smem[row, k]
                descs.append(
                    pltpu.make_async_copy(
                        x_hbm.at[row, pl.ds(cid * _CW, _CW)],
                        cand.at[slot, r, pl.ds(k * _CW, _CW)],
                        sems.at[slot]))
        return descs

    even = pid % 2 == 0

    @pl.when(pid == 0)
    def _():
        for d in chunk_descs(0, 0):
            d.start()

    @pl.when((pid + 1 < ngrid) & even)
    def _():
        for d in chunk_descs(pid + 1, 1):
            d.start()

    @pl.when((pid + 1 < ngrid) & ~even)
    def _():
        for d in chunk_descs(pid + 1, 0):
            d.start()

    def drain(slot):
        pltpu.make_async_copy(
            x_hbm.at[pl.ds(0, _BRC), pl.ds(0, _CAND)],
            cand.at[slot], sems.at[slot]).wait()

    @pl.when(even)
    def _():
        drain(0)

    @pl.when(~even)
    def _():
        drain(1)

    g = jnp.where(even, cand[0], cand[1])
    m = ms_ref[:, 0:1]
    s = ms_ref[:, 16:17]
    cids = cids_ref[...]

    nl = _CAND // 128
    xm = g.reshape(_BRC, nl, 128)
    gidx = (jax.lax.broadcasted_iota(jnp.int32, (1, nl, 128), 1) * 128
            + jax.lax.broadcasted_iota(jnp.int32, (1, nl, 128), 2))
    lane = jax.lax.broadcasted_iota(jnp.int32, (_BRC, 128), 1)
    big = jnp.int32(2**30)

    M = jnp.max(xm, axis=1)

    vals = []
    idxs = []
    for _ in range(_K):
        mk = jnp.max(M, axis=1, keepdims=True)
        hit = xm == mk[:, :, None]
        a = jnp.min(
            jnp.where(hit,
                      jax.lax.broadcasted_iota(jnp.int32, (1, nl, 128), 1),
                      big),
            axis=1)
        cand_i = jnp.where(a < nl, a * 128 + lane, big)
        ik = jnp.min(cand_i, axis=1, keepdims=True)
        vals.append(mk)
        idxs.append(ik)
        xm = jnp.where(gidx == ik[:, :, None], _NEG, xm)
        M = jnp.max(xm, axis=1)

    v = jnp.concatenate(vals, axis=1)
    li = jnp.concatenate(idxs, axis=1)
    slot = li // _CW
    within = li % _CW
    cid_sel = jnp.zeros_like(li)
    for k in range(_K):
        cid_sel = jnp.where(slot == k, cids[:, k:k + 1], cid_sel)
    idx_ref[...] = cid_sel * _CW + within
    vals_ref[...] = jnp.exp(v - m) / s


def _tc_final(x, cids, ms):
    grid_spec = pltpu.PrefetchScalarGridSpec(
        num_scalar_prefetch=1,
        grid=(_ROWS // _BRC,),
        in_specs=[
            pl.BlockSpec(memory_space=pl.ANY),
            pl.BlockSpec((_BRC, _K), lambda i, s_ref: (i, 0)),
            pl.BlockSpec((_BRC, 32), lambda i, s_ref: (i, 0)),
        ],
        out_specs=[
            pl.BlockSpec((_BRC, _K), lambda i, s_ref: (i, 0)),
            pl.BlockSpec((_BRC, _K), lambda i, s_ref: (i, 0)),
        ],
        scratch_shapes=[
            pltpu.VMEM((2, _BRC, _CAND), jnp.float32),
            pltpu.SemaphoreType.DMA((2,)),
        ],
    )
    return pl.pallas_call(
        _final_body,
        grid_spec=grid_spec,
        out_shape=[
            jax.ShapeDtypeStruct((_ROWS, _K), jnp.float32),
            jax.ShapeDtypeStruct((_ROWS, _K), jnp.int32),
        ],
    )(cids, x, cids, ms)


@jax.jit
def kernel(x):
    cm = _tc_cm(x)
    (cids,) = _sc_select(cm.reshape(-1))
    ms = _tc_ms(x, cm)
    vals, idx = _tc_final(x, cids.reshape(_ROWS, _K), ms)
    return vals, idx


# speedup vs baseline: 1.0701x; 1.0701x over previous
import functools

import jax
import jax.numpy as jnp
from jax import lax
from jax.experimental import pallas as pl
from jax.experimental.pallas import tpu as pltpu
from jax.experimental.pallas import tpu_sc as plsc

_ROWS = 128
_COLS = 32768
_K = 8
_BLOCK_ROWS = 16
_NCH = 128
_CW = 256
_CAND = _K * _CW
_NEG = float("-inf")
_BIGI = 2**30

_NC = 2
_NS = 16
_RPW = _ROWS // (_NC * _NS)


def _stats_body(x_ref, cm_ref, ms_ref):
    x = x_ref[...]
    cm = jnp.max(x.reshape(_BLOCK_ROWS, _NCH, _CW), axis=2)
    m = jnp.max(cm, axis=1, keepdims=True)
    s = jnp.sum(jnp.exp(x - m), axis=1, keepdims=True)
    cm_ref[...] = cm
    ms_ref[...] = jnp.concatenate(
        [jnp.broadcast_to(m, (_BLOCK_ROWS, 16)),
         jnp.broadcast_to(s, (_BLOCK_ROWS, 16))], axis=1)


def _tc_stats(x):
    return pl.pallas_call(
        _stats_body,
        grid=(_ROWS // _BLOCK_ROWS,),
        in_specs=[pl.BlockSpec((_BLOCK_ROWS, _COLS), lambda i: (i, 0))],
        out_specs=[
            pl.BlockSpec((_BLOCK_ROWS, _NCH), lambda i: (i, 0)),
            pl.BlockSpec((_BLOCK_ROWS, 32), lambda i: (i, 0)),
        ],
        out_shape=[
            jax.ShapeDtypeStruct((_ROWS, _NCH), jnp.float32),
            jax.ShapeDtypeStruct((_ROWS, 32), jnp.float32),
        ],
    )(x)


def _iota16():
    return lax.broadcasted_iota(jnp.int32, (16,), 0)


_GDN = lax.GatherDimensionNumbers(
    offset_dims=(), collapsed_slice_dims=(0,), start_index_map=(0,))


def _perm(v, s):
    idx = (_iota16() ^ s).reshape(16, 1)
    return lax.gather(v, idx, _GDN, (1,),
                      mode=lax.GatherScatterMode.PROMISE_IN_BOUNDS)


def _bfly_max(v):
    for s in (1, 2, 4, 8):
        v = jnp.maximum(v, _perm(v, s))
    return v


def _bfly_min(v):
    for s in (1, 2, 4, 8):
        v = jnp.minimum(v, _perm(v, s))
    return v


def _sc_body(cm_hbm, cids_hbm, statsbuf, ibuf):
    iota = _iota16()
    wid = lax.axis_index("s") * _NC + lax.axis_index("c")
    row0 = wid * _RPW

    pltpu.sync_copy(cm_hbm.at[pl.ds(row0 * _NCH, _RPW * _NCH)], statsbuf)

    nv = _NCH // 16

    srts = []
    for rr in range(_RPW):
        cms = [statsbuf[pl.ds(rr * _NCH + 16 * q, 16)] for q in range(nv)]
        cids_vec = jnp.full((16,), _BIGI, jnp.int32)
        for k in range(_K):
            e = cms[0]
            for q in range(1, nv):
                e = jnp.maximum(e, cms[q])
            g = _bfly_max(e)
            im = jnp.full((16,), 4 * _NCH, jnp.int32)
            for q in range(nv):
                im = jnp.minimum(im, jnp.where(cms[q] == g, iota + 16 * q,
                                               4 * _NCH))
            cid = _bfly_min(im)
            for q in range(nv):
                cms[q] = jnp.where(iota + 16 * q == cid, _NEG, cms[q])
            cids_vec = jnp.where(iota == k, cid, cids_vec)
        srt = jnp.full((16,), _BIGI, jnp.int32)
        for k in range(_K):
            c = _bfly_min(cids_vec)
            srt = jnp.where(iota == k, c, srt)
            cids_vec = jnp.where(cids_vec == c, _BIGI, cids_vec)
        srts.append(srt)

    for pair in range(_RPW // 2):
        lo, hi = srts[2 * pair], srts[2 * pair + 1]
        both = jnp.where(iota < 8, lo, _perm(hi, 8))
        ibuf[pl.ds(pair * 16, 16)] = both
    pltpu.sync_copy(ibuf, cids_hbm.at[pl.ds(row0 * _K, _RPW * _K)])


@functools.partial(
    pl.kernel,
    out_type=[jax.ShapeDtypeStruct((_ROWS * _K,), jnp.int32)],
    mesh=plsc.VectorSubcoreMesh(core_axis_name="c", subcore_axis_name="s"),
    scratch_types=[
        pltpu.VMEM((_RPW * _NCH,), jnp.float32),
        pltpu.VMEM((_RPW * _K,), jnp.int32),
    ],
)
def _sc_select(cm_hbm, cids_hbm, statsbuf, ibuf):
    _sc_body(cm_hbm, cids_hbm, statsbuf, ibuf)


_BRC = 16


def _final_body(cids_smem, x_hbm, cids_ref, ms_ref, vals_ref, idx_ref,
                cand, sems):
    pid = pl.program_id(0)
    ngrid = _ROWS // _BRC

    def chunk_descs(blk, slot):
        descs = []
        for r in range(_BRC):
            row = blk * _BRC + r
            for k in range(_K):
                cid = cids_smem[row, k]
                descs.append(
                    pltpu.make_async_copy(
                        x_hbm.at[row, pl.ds(cid * _CW, _CW)],
                        cand.at[slot, r, pl.ds(k * _CW, _CW)],
                        sems.at[slot]))
        return descs

    even = pid % 2 == 0

    @pl.when(pid == 0)
    def _():
        for d in chunk_descs(0, 0):
            d.start()

    @pl.when((pid + 1 < ngrid) & even)
    def _():
        for d in chunk_descs(pid + 1, 1):
            d.start()

    @pl.when((pid + 1 < ngrid) & ~even)
    def _():
        for d in chunk_descs(pid + 1, 0):
            d.start()

    def drain(slot):
        pltpu.make_async_copy(
            x_hbm.at[pl.ds(0, _BRC), pl.ds(0, _CAND)],
            cand.at[slot], sems.at[slot]).wait()

    @pl.when(even)
    def _():
        drain(0)

    @pl.when(~even)
    def _():
        drain(1)

    g = jnp.where(even, cand[0], cand[1])
    m = ms_ref[:, 0:1]
    s = ms_ref[:, 16:17]
    cids = cids_ref[...]

    nl = _CAND // 128
    xm = g.reshape(_BRC, nl, 128)
    gidx = (jax.lax.broadcasted_iota(jnp.int32, (1, nl, 128), 1) * 128
            + jax.lax.broadcasted_iota(jnp.int32, (1, nl, 128), 2))
    lane = jax.lax.broadcasted_iota(jnp.int32, (_BRC, 128), 1)
    big = jnp.int32(2**30)

    M = jnp.max(xm, axis=1)

    vals = []
    idxs = []
    for _ in range(_K):
        mk = jnp.max(M, axis=1, keepdims=True)
        hit = xm == mk[:, :, None]
        a = jnp.min(
            jnp.where(hit,
                      jax.lax.broadcasted_iota(jnp.int32, (1, nl, 128), 1),
                      big),
            axis=1)
        cand_i = jnp.where(a < nl, a * 128 + lane, big)
        ik = jnp.min(cand_i, axis=1, keepdims=True)
        vals.append(mk)
        idxs.append(ik)
        xm = jnp.where(gidx == ik[:, :, None], _NEG, xm)
        M = jnp.max(xm, axis=1)

    v = jnp.concatenate(vals, axis=1)
    li = jnp.concatenate(idxs, axis=1)
    slot = li // _CW
    within = li % _CW
    cid_sel = jnp.zeros_like(li)
    for k in range(_K):
        cid_sel = jnp.where(slot == k, cids[:, k:k + 1], cid_sel)
    idx_ref[...] = cid_sel * _CW + within
    vals_ref[...] = jnp.exp(v - m) / s


def _tc_final(x, cids, ms):
    grid_spec = pltpu.PrefetchScalarGridSpec(
        num_scalar_prefetch=1,
        grid=(_ROWS // _BRC,),
        in_specs=[
            pl.BlockSpec(memory_space=pl.ANY),
            pl.BlockSpec((_BRC, _K), lambda i, s_ref: (i, 0)),
            pl.BlockSpec((_BRC, 32), lambda i, s_ref: (i, 0)),
        ],
        out_specs=[
            pl.BlockSpec((_BRC, _K), lambda i, s_ref: (i, 0)),
            pl.BlockSpec((_BRC, _K), lambda i, s_ref: (i, 0)),
        ],
        scratch_shapes=[
            pltpu.VMEM((2, _BRC, _CAND), jnp.float32),
            pltpu.SemaphoreType.DMA((2,)),
        ],
    )
    return pl.pallas_call(
        _final_body,
        grid_spec=grid_spec,
        out_shape=[
            jax.ShapeDtypeStruct((_ROWS, _K), jnp.float32),
            jax.ShapeDtypeStruct((_ROWS, _K), jnp.int32),
        ],
    )(cids, x, cids, ms)


@jax.jit
def kernel(x):
    cm, ms = _tc_stats(x)
    (cids,) = _sc_select(cm.reshape(-1))
    vals, idx = _tc_final(x, cids.reshape(_ROWS, _K), ms)
    return vals, idx
